# baseline (speedup 1.0000x reference)
import jax
import jax.numpy as jnp
from jax import lax
from jax.experimental import pallas as pl
from jax.experimental.pallas import tpu as pltpu
from jax.experimental.pallas import tpu_sc as plsc

NC, NS, L = 2, 16, 16
NW = NC * NS
B, SEQ, T, D = 1024, 50, 26, 32
PLANES = SEQ * T
PPW = (PLANES + NW - 1) // NW
PLANE = D * B
NBUF = 3


def _body(xt_hbm, tab_hbm, out_hbm, xs_v, t2_v, rep_v, stg_v, sem):
    wid = lax.axis_index("s") * NC + lax.axis_index("c")

    pltpu.sync_copy(tab_hbm.at[pl.ds(0, 2 * D)], t2_v)
    halves = [t2_v[pl.ds(h * L, L)] for h in range(4)]

    for i in range(D):
        rep_v[pl.ds(i * L, L)] = jnp.full((L,), halves[i // L][i % L])
        rep_v[pl.ds((D + i) * L, L)] = jnp.full((L,), halves[2 + i // L][i % L])

    def build_plane(t, bo):
        for dt in range(4):
            r0 = [rep_v[pl.ds((dt * 8 + r) * L, L)] for r in range(8)]
            r1 = [rep_v[pl.ds((D + dt * 8 + r) * L, L)] for r in range(8)]

            def chunk(k, _, dt=dt, r0=r0, r1=r1):
                xv = xs_v[pl.ds(k * L, L)]
                mk = xv == t
                base = bo + (dt * 8 + k // 8) * B + (k % 8) * L
                for r in range(8):
                    stg_v[pl.ds(base + r * 128, L)] = jnp.where(mk, r1[r], r0[r])
                return _
            lax.fori_loop(0, B // L, chunk, None, unroll=4)

    nP = jnp.where(wid < PLANES - (PPW - 1) * NW, PPW, PPW - 1)

    def step(i, _):
        p = wid + i * NW

        @pl.when(jnp.logical_and(i >= NBUF, i - NBUF < nP))
        def _wait():
            pltpu.make_async_copy(
                stg_v.at[pl.ds(0, PLANE)],
                out_hbm.at[pl.ds(0, PLANE)],
                sem).wait()

        @pl.when(i < nP)
        def _work():
            s = p // T
            t = p % T
            pltpu.sync_copy(xt_hbm.at[pl.ds(s * B, B)], xs_v)
            b = lax.rem(i, NBUF)
            build_plane(t, b * PLANE)
            pltpu.async_copy(
                stg_v.at[pl.ds(b * PLANE, PLANE)],
                out_hbm.at[pl.ds(p * PLANE, PLANE)],
                sem)
        return _
    lax.fori_loop(0, PPW + NBUF, step, None)


@jax.jit
def _run(xt_flat, tab_flat):
    mesh = plsc.VectorSubcoreMesh(
        core_axis_name="c", subcore_axis_name="s",
        num_cores=NC, num_subcores=NS)
    return pl.kernel(
        _body,
        out_type=jax.ShapeDtypeStruct((PLANES * PLANE,), jnp.float32),
        mesh=mesh,
        scratch_types=[
            pltpu.VMEM((B,), jnp.int32),
            pltpu.VMEM((2 * D,), jnp.float32),
            pltpu.VMEM((2 * D * L,), jnp.float32),
            pltpu.VMEM((NBUF * PLANE,), jnp.float32),
            pltpu.SemaphoreType.DMA,
        ],
    )(xt_flat, tab_flat)


def kernel(x, table):
    xt = x.T.reshape(-1)
    flat = _run(xt, table.reshape(-1))
    o6 = flat.reshape(SEQ, T, 4, 8, 8, 128)
    z = o6.transpose(3, 5, 0, 1, 2, 4)
    return z.reshape(B, SEQ, T, D)


# speedup vs baseline: 1.2793x; 1.2793x over previous
import jax
import jax.numpy as jnp
from jax import lax
from jax.experimental import pallas as pl
from jax.experimental.pallas import tpu as pltpu
from jax.experimental.pallas import tpu_sc as plsc

NC, NS, L = 2, 16, 16
NW = NC * NS
B, SEQ, T, D = 1024, 50, 26, 32
PLANES = SEQ * T
PPW = (PLANES + NW - 1) // NW
PLANE = D * B
NBUF = 3


def _body(xt_hbm, tab_hbm, out_hbm, xs_v, t2_v, rep_v, stg_v, sem, sem_x):
    wid = lax.axis_index("s") * NC + lax.axis_index("c")

    pltpu.sync_copy(tab_hbm.at[pl.ds(0, 2 * D)], t2_v)
    halves = [t2_v[pl.ds(h * L, L)] for h in range(4)]

    for i in range(D):
        rep_v[pl.ds(i * L, L)] = jnp.full((L,), halves[i // L][i % L])
        rep_v[pl.ds((D + i) * L, L)] = jnp.full((L,), halves[2 + i // L][i % L])

    def build_plane(t, bo, xo):
        for dt in range(4):
            r0 = [rep_v[pl.ds((dt * 8 + r) * L, L)] for r in range(8)]
            r1 = [rep_v[pl.ds((D + dt * 8 + r) * L, L)] for r in range(8)]

            def chunk(k, _, dt=dt, r0=r0, r1=r1):
                xv = xs_v[pl.ds(xo + k * L, L)]
                mk = xv == t
                base = bo + (dt * 8 + k // 8) * B + (k % 8) * L
                for r in range(8):
                    stg_v[pl.ds(base + r * 128, L)] = jnp.where(mk, r1[r], r0[r])
                return _
            lax.fori_loop(0, B // L, chunk, None, unroll=4)

    nP = jnp.where(wid < PLANES - (PPW - 1) * NW, PPW, PPW - 1)

    pltpu.async_copy(xt_hbm.at[pl.ds((wid // T) * B, B)],
                     xs_v.at[pl.ds(0, B)], sem_x)

    def step(i, _):
        p = wid + i * NW

        @pl.when(jnp.logical_and(i >= NBUF, i - NBUF < nP))
        def _wait():
            pltpu.make_async_copy(
                stg_v.at[pl.ds(0, PLANE)],
                out_hbm.at[pl.ds(0, PLANE)],
                sem).wait()

        @pl.when(i < nP)
        def _work():
            xo = lax.rem(i, 2) * B
            pltpu.make_async_copy(
                xt_hbm.at[pl.ds(0, B)], xs_v.at[pl.ds(0, B)], sem_x).wait()

            @pl.when(i + 1 < nP)
            def _prefetch():
                s_next = (p + NW) // T
                pltpu.async_copy(
                    xt_hbm.at[pl.ds(s_next * B, B)],
                    xs_v.at[pl.ds((B - xo) % (2 * B), B)], sem_x)

            t = p % T
            b = lax.rem(i, NBUF)
            build_plane(t, b * PLANE, xo)
            pltpu.async_copy(
                stg_v.at[pl.ds(b * PLANE, PLANE)],
                out_hbm.at[pl.ds(p * PLANE, PLANE)],
                sem)
        return _
    lax.fori_loop(0, PPW + NBUF, step, None)


@jax.jit
def _run(xt_flat, tab_flat):
    mesh = plsc.VectorSubcoreMesh(
        core_axis_name="c", subcore_axis_name="s",
        num_cores=NC, num_subcores=NS)
    return pl.kernel(
        _body,
        out_type=jax.ShapeDtypeStruct((PLANES * PLANE,), jnp.float32),
        mesh=mesh,
        scratch_types=[
            pltpu.VMEM((2 * B,), jnp.int32),
            pltpu.VMEM((2 * D,), jnp.float32),
            pltpu.VMEM((2 * D * L,), jnp.float32),
            pltpu.VMEM((NBUF * PLANE,), jnp.float32),
            pltpu.SemaphoreType.DMA,
            pltpu.SemaphoreType.DMA,
        ],
    )(xt_flat, tab_flat)


def kernel(x, table):
    xt = x.T.reshape(-1)
    flat = _run(xt, table.reshape(-1))
    o6 = flat.reshape(SEQ, T, 4, 8, 8, 128)
    z = o6.transpose(3, 5, 0, 1, 2, 4)
    return z.reshape(B, SEQ, T, D)


# speedup vs baseline: 1.4273x; 1.1157x over previous
import jax
import jax.numpy as jnp
from jax import lax
from jax.experimental import pallas as pl
from jax.experimental.pallas import tpu as pltpu
from jax.experimental.pallas import tpu_sc as plsc

NC, NS, L = 2, 16, 16
NW = NC * NS
B, SEQ, T, D = 1024, 50, 26, 32
PLANES = SEQ * T
PPW = (PLANES + NW - 1) // NW
PLANE = D * B
NBUF = 3


def _body(xt_hbm, tab_hbm, out_hbm, xs_v, t2_v, rep_v, stg_v, sem, sem_x):
    wid = lax.axis_index("s") * NC + lax.axis_index("c")

    pltpu.sync_copy(tab_hbm.at[pl.ds(0, 2 * D)], t2_v)
    halves = [t2_v[pl.ds(h * L, L)] for h in range(4)]

    for i in range(D):
        rep_v[pl.ds(i * L, L)] = jnp.full((L,), halves[i // L][i % L])
        rep_v[pl.ds((D + i) * L, L)] = jnp.full((L,), halves[2 + i // L][i % L])

    def build_plane(t, bo, xo):
        for dth in range(2):
            r0 = [rep_v[pl.ds((dth * 16 + j) * L, L)] for j in range(16)]
            r1 = [rep_v[pl.ds((D + dth * 16 + j) * L, L)] for j in range(16)]

            def chunk(k, _, dth=dth, r0=r0, r1=r1):
                xv = xs_v[pl.ds(xo + k * L, L)]
                mk = xv == t
                base = bo + (k // 8) * B + (k % 8) * L
                for j in range(16):
                    off = base + (dth * 16 + (j // 8) * 8) * B + (j % 8) * 128
                    stg_v[pl.ds(off, L)] = jnp.where(mk, r1[j], r0[j])
                return _
            lax.fori_loop(0, B // L, chunk, None, unroll=4)

    nP = jnp.where(wid < PLANES - (PPW - 1) * NW, PPW, PPW - 1)

    pltpu.async_copy(xt_hbm.at[pl.ds((wid // T) * B, B)],
                     xs_v.at[pl.ds(0, B)], sem_x)

    def step(i, _):
        p = wid + i * NW

        @pl.when(jnp.logical_and(i >= NBUF, i - NBUF < nP))
        def _wait():
            pltpu.make_async_copy(
                stg_v.at[pl.ds(0, PLANE)],
                out_hbm.at[pl.ds(0, PLANE)],
                sem).wait()

        @pl.when(i < nP)
        def _work():
            xo = lax.rem(i, 2) * B
            pltpu.make_async_copy(
                xt_hbm.at[pl.ds(0, B)], xs_v.at[pl.ds(0, B)], sem_x).wait()

            @pl.when(i + 1 < nP)
            def _prefetch():
                s_next = (p + NW) // T
                pltpu.async_copy(
                    xt_hbm.at[pl.ds(s_next * B, B)],
                    xs_v.at[pl.ds((B - xo) % (2 * B), B)], sem_x)

            t = p % T
            b = lax.rem(i, NBUF)
            build_plane(t, b * PLANE, xo)
            pltpu.async_copy(
                stg_v.at[pl.ds(b * PLANE, PLANE)],
                out_hbm.at[pl.ds(p * PLANE, PLANE)],
                sem)
        return _
    lax.fori_loop(0, PPW + NBUF, step, None)


@jax.jit
def _run(xt_flat, tab_flat):
    mesh = plsc.VectorSubcoreMesh(
        core_axis_name="c", subcore_axis_name="s",
        num_cores=NC, num_subcores=NS)
    return pl.kernel(
        _body,
        out_type=jax.ShapeDtypeStruct((PLANES * PLANE,), jnp.float32),
        mesh=mesh,
        scratch_types=[
            pltpu.VMEM((2 * B,), jnp.int32),
            pltpu.VMEM((2 * D,), jnp.float32),
            pltpu.VMEM((2 * D * L,), jnp.float32),
            pltpu.VMEM((NBUF * PLANE,), jnp.float32),
            pltpu.SemaphoreType.DMA,
            pltpu.SemaphoreType.DMA,
        ],
    )(xt_flat, tab_flat)


def kernel(x, table):
    xt = x.T.reshape(-1)
    flat = _run(xt, table.reshape(-1))
    o6 = flat.reshape(SEQ, T, 4, 8, 8, 128)
    z = o6.transpose(3, 5, 0, 1, 2, 4)
    return z.reshape(B, SEQ, T, D)


# speedup vs baseline: 1.4439x; 1.0116x over previous
import jax
import jax.numpy as jnp
from jax import lax
from jax.experimental import pallas as pl
from jax.experimental.pallas import tpu as pltpu
from jax.experimental.pallas import tpu_sc as plsc

NC, NS, L = 2, 16, 16
NW = NC * NS
B, SEQ, T, D = 1024, 50, 26, 32
PLANES = SEQ * T
PPW = (PLANES + NW - 1) // NW
PLANE = D * B
NBUF = 3


def _body(xt_hbm, tab_hbm, out_hbm, xs_v, t2_v, rep_v, stg_v, sem, sem_x):
    wid = lax.axis_index("s") * NC + lax.axis_index("c")

    pltpu.sync_copy(tab_hbm.at[pl.ds(0, 2 * D)], t2_v)
    halves = [t2_v[pl.ds(h * L, L)] for h in range(4)]

    for i in range(D):
        rep_v[pl.ds(i * L, L)] = jnp.full((L,), halves[i // L][i % L])
        rep_v[pl.ds((D + i) * L, L)] = jnp.full((L,), halves[2 + i // L][i % L])

    def build_plane(t, bo, xo):
        for dth in range(2):
            r0 = [rep_v[pl.ds((dth * 16 + j) * L, L)] for j in range(16)]
            r1 = [rep_v[pl.ds((D + dth * 16 + j) * L, L)] for j in range(16)]

            def chunk(k, _, dth=dth, r0=r0, r1=r1):
                xv = xs_v[pl.ds(xo + k * L, L)]
                mk = xv == t
                base = bo + (k // 8) * B + (k % 8) * L
                for j in range(16):
                    off = base + (dth * 16 + (j // 8) * 8) * B + (j % 8) * 128
                    stg_v[pl.ds(off, L)] = jnp.where(mk, r1[j], r0[j])
                return _
            lax.fori_loop(0, B // L, chunk, None, unroll=8)

    nP = jnp.where(wid < PLANES - (PPW - 1) * NW, PPW, PPW - 1)

    pltpu.async_copy(xt_hbm.at[pl.ds((wid // T) * B, B)],
                     xs_v.at[pl.ds(0, B)], sem_x)

    def step(i, _):
        p = wid + i * NW

        @pl.when(jnp.logical_and(i >= NBUF, i - NBUF < nP))
        def _wait():
            pltpu.make_async_copy(
                stg_v.at[pl.ds(0, PLANE)],
                out_hbm.at[pl.ds(0, PLANE)],
                sem).wait()

        @pl.when(i < nP)
        def _work():
            xo = lax.rem(i, 2) * B
            pltpu.make_async_copy(
                xt_hbm.at[pl.ds(0, B)], xs_v.at[pl.ds(0, B)], sem_x).wait()

            @pl.when(i + 1 < nP)
            def _prefetch():
                s_next = (p + NW) // T
                pltpu.async_copy(
                    xt_hbm.at[pl.ds(s_next * B, B)],
                    xs_v.at[pl.ds((B - xo) % (2 * B), B)], sem_x)

            t = p % T
            b = lax.rem(i, NBUF)
            build_plane(t, b * PLANE, xo)
            pltpu.async_copy(
                stg_v.at[pl.ds(b * PLANE, PLANE)],
                out_hbm.at[pl.ds(p * PLANE, PLANE)],
                sem)
        return _
    lax.fori_loop(0, PPW + NBUF, step, None)


@jax.jit
def _run(xt_flat, tab_flat):
    mesh = plsc.VectorSubcoreMesh(
        core_axis_name="c", subcore_axis_name="s",
        num_cores=NC, num_subcores=NS)
    return pl.kernel(
        _body,
        out_type=jax.ShapeDtypeStruct((PLANES * PLANE,), jnp.float32),
        mesh=mesh,
        scratch_types=[
            pltpu.VMEM((2 * B,), jnp.int32),
            pltpu.VMEM((2 * D,), jnp.float32),
            pltpu.VMEM((2 * D * L,), jnp.float32),
            pltpu.VMEM((NBUF * PLANE,), jnp.float32),
            pltpu.SemaphoreType.DMA,
            pltpu.SemaphoreType.DMA,
        ],
    )(xt_flat, tab_flat)


def kernel(x, table):
    xt = x.T.reshape(-1)
    flat = _run(xt, table.reshape(-1))
    o6 = flat.reshape(SEQ, T, 4, 8, 8, 128)
    z = o6.transpose(3, 5, 0, 1, 2, 4)
    return z.reshape(B, SEQ, T, D)
